# baseline (speedup 1.0000x reference)
import functools
import math

import jax
import jax.numpy as jnp
from jax import lax
from jax.experimental import pallas as pl
from jax.experimental.pallas import tpu as pltpu
from jax.experimental.pallas import tpu_sc as plsc

D_MODEL = 1024
LANES = 16
NUM_WORKERS = 32
CHUNK = 32


def _emb_body(seq_len, rows_per_worker, ids_hbm, table_hbm, pe_hbm, out_hbm,
              idx_v, tok_v, pe_v, sem):
    wid = lax.axis_index("s") * 2 + lax.axis_index("c")
    base = wid * rows_per_worker
    pltpu.sync_copy(ids_hbm.at[pl.ds(base, rows_per_worker)], idx_v)

    n_chunks = rows_per_worker // CHUNK
    vregs_per_chunk = CHUNK * D_MODEL // LANES
    cols = D_MODEL // LANES

    def chunk_body(k, _):
        row0 = base + k * CHUNK
        pos0 = lax.rem(row0, seq_len)
        pltpu.sync_copy(pe_hbm.at[pl.ds(pos0, CHUNK)], pe_v)
        pltpu.async_copy(table_hbm.at[idx_v.at[pl.ds(k * CHUNK, CHUNK)]],
                         tok_v, sem).wait()

        def vec_body(i, _):
            r = i // cols
            c = (i % cols) * LANES
            tok_v[r, pl.ds(c, LANES)] = (
                tok_v[r, pl.ds(c, LANES)] * math.sqrt(D_MODEL)
                + pe_v[r, pl.ds(c, LANES)])
            return 0

        lax.fori_loop(0, vregs_per_chunk, vec_body, 0)
        pltpu.sync_copy(tok_v, out_hbm.at[pl.ds(row0, CHUNK)])
        return 0

    lax.fori_loop(0, n_chunks, chunk_body, 0)


@jax.jit
def kernel(token_ids, W_tok, pe):
    batch, seq_len = token_ids.shape
    n_rows = batch * seq_len
    rows_per_worker = n_rows // NUM_WORKERS
    ids = token_ids.reshape(-1).astype(jnp.int32)

    mesh = plsc.VectorSubcoreMesh(core_axis_name="c", subcore_axis_name="s")
    body = functools.partial(_emb_body, seq_len, rows_per_worker)
    out = pl.kernel(
        body,
        mesh=mesh,
        out_type=jax.ShapeDtypeStruct((n_rows, D_MODEL), jnp.float32),
        scratch_types=[
            pltpu.VMEM((rows_per_worker,), jnp.int32),
            pltpu.VMEM((CHUNK, D_MODEL), jnp.float32),
            pltpu.VMEM((CHUNK, D_MODEL), jnp.float32),
            pltpu.SemaphoreType.DMA,
        ],
    )(ids, W_tok, pe)
    return out.reshape(batch, seq_len, D_MODEL)


# speedup vs baseline: 3.0782x; 3.0782x over previous
import functools
import math

import jax
import jax.numpy as jnp
from jax import lax
from jax.experimental import pallas as pl
from jax.experimental.pallas import tpu as pltpu
from jax.experimental.pallas import tpu_sc as plsc

D_MODEL = 1024
LANES = 16
NUM_WORKERS = 32
CHUNK = 16
SCALE = math.sqrt(D_MODEL)


def _emb_body(seq_len, rows_per_worker, ids_hbm, table_hbm, pe_hbm, out_hbm,
              idx_v, tok0, tok1, acc0, acc1, acc2, acc3,
              st0, st1, sp0, sp1, sp2, sp3, sw0, sw1, sw2, sw3):
    toks = (tok0, tok1)
    accs = (acc0, acc1, acc2, acc3)
    sts = (st0, st1)
    sps = (sp0, sp1, sp2, sp3)
    sws = (sw0, sw1, sw2, sw3)

    steps = rows_per_worker // CHUNK
    wid = lax.axis_index("s") * 2 + lax.axis_index("c")
    base = wid * rows_per_worker
    pltpu.sync_copy(ids_hbm.at[pl.ds(base, rows_per_worker)], idx_v)

    def pe_copy(g, ab):
        pos0 = lax.rem(base + g * CHUNK, seq_len)
        return pltpu.make_async_copy(
            pe_hbm.at[pl.ds(pos0, CHUNK)], accs[ab], sps[ab])

    def gather_copy(g, tb):
        return pltpu.make_async_copy(
            table_hbm.at[idx_v.at[pl.ds(g * CHUNK, CHUNK)]], toks[tb], sts[tb])

    def write_copy(g, ab):
        return pltpu.make_async_copy(
            accs[ab], out_hbm.at[pl.ds(base + g * CHUNK, CHUNK)], sws[ab])

    for j in (0, 1):
        pe_copy(j, j).start()
        gather_copy(j, j).start()

    def outer(i, _):
        for j in range(4):
            g = i * 4 + j
            ab, tb = j, j % 2
            gather_copy(g, tb).wait()
            pe_copy(g, ab).wait()

            def rows(r, _):
                for c in range(D_MODEL // LANES):
                    v = toks[tb][r, pl.ds(c * LANES, LANES)]
                    plsc.addupdate(accs[ab].at[r, pl.ds(c * LANES, LANES)],
                                   v * SCALE)
                return 0

            lax.fori_loop(0, CHUNK, rows, 0)
            write_copy(g, ab).start()

            nab = (j + 2) % 4

            @pl.when(g + 2 < steps)
            def _():
                @pl.when(g >= 2)
                def _():
                    write_copy(g - 2, nab).wait()
                pe_copy(g + 2, nab).start()
                gather_copy(g + 2, tb).start()
        return 0

    lax.fori_loop(0, steps // 4, outer, 0)

    for j in range(4):
        write_copy(steps - 4 + j, j).wait()


@jax.jit
def kernel(token_ids, W_tok, pe):
    batch, seq_len = token_ids.shape
    n_rows = batch * seq_len
    rows_per_worker = n_rows // NUM_WORKERS
    ids = token_ids.reshape(-1).astype(jnp.int32)

    mesh = plsc.VectorSubcoreMesh(core_axis_name="c", subcore_axis_name="s")
    body = functools.partial(_emb_body, seq_len, rows_per_worker)
    out = pl.kernel(
        body,
        mesh=mesh,
        out_type=jax.ShapeDtypeStruct((n_rows, D_MODEL), jnp.float32),
        scratch_types=(
            [pltpu.VMEM((rows_per_worker,), jnp.int32)]
            + [pltpu.VMEM((CHUNK, D_MODEL), jnp.float32) for _ in range(6)]
            + [pltpu.SemaphoreType.DMA for _ in range(10)]
        ),
    )(ids, W_tok, pe)
    return out.reshape(batch, seq_len, D_MODEL)


# speedup vs baseline: 3.2469x; 1.0548x over previous
import functools
import math

import jax
import jax.numpy as jnp
from jax import lax
from jax.experimental import pallas as pl
from jax.experimental.pallas import tpu as pltpu
from jax.experimental.pallas import tpu_sc as plsc

D_MODEL = 1024
LANES = 16
NUM_WORKERS = 32
CHUNK = 16
SCALE = math.sqrt(D_MODEL)


def _emb_body(batch, seq_len, ids_hbm, table_hbm, pe_hbm, out_hbm,
              idx_v, tok0, tok1, tok2, tok3, pe0, pe1,
              st0, st1, st2, st3, sw0, sw1, sw2, sw3, sp0, sp1):
    toks = (tok0, tok1, tok2, tok3)
    pes = (pe0, pe1)
    sts = (st0, st1, st2, st3)
    sws = (sw0, sw1, sw2, sw3)
    sps = (sp0, sp1)

    pos_per_worker = seq_len // NUM_WORKERS
    n_pc = pos_per_worker // CHUNK
    steps = n_pc * batch

    wid = lax.axis_index("s") * 2 + lax.axis_index("c")
    wpos = wid * pos_per_worker

    for b in range(batch):
        pltpu.sync_copy(ids_hbm.at[pl.ds(b * seq_len + wpos, pos_per_worker)],
                        idx_v.at[pl.ds(b * pos_per_worker, pos_per_worker)])

    def gather_copy(pc, bb, tb):
        return pltpu.make_async_copy(
            table_hbm.at[idx_v.at[pl.ds(bb * pos_per_worker + pc * CHUNK,
                                        CHUNK)]],
            toks[tb], sts[tb])

    def pe_copy(pc, pb):
        return pltpu.make_async_copy(
            pe_hbm.at[pl.ds(wpos + pc * CHUNK, CHUNK)], pes[pb], sps[pb])

    def write_copy(pc, bb, tb):
        return pltpu.make_async_copy(
            toks[tb], out_hbm.at[pl.ds(bb * seq_len + wpos + pc * CHUNK,
                                       CHUNK)], sws[tb])

    pe_copy(0, 0).start()
    gather_copy(0, 0, 0).start()
    gather_copy(0, 1, 1).start()

    def outer(i, _):
        for j in range(2 * batch):
            pc = 2 * i + j // batch
            bb = j % batch
            g = 2 * batch * i + j
            tb = j % 4
            pb = (j // batch) % 2

            gather_copy(pc, bb, tb).wait()
            if bb == 0:
                pe_copy(pc, pb).wait()

            def rows(r, _):
                for c in range(D_MODEL // LANES):
                    sl = pl.ds(c * LANES, LANES)
                    toks[tb][r, sl] = toks[tb][r, sl] * SCALE + pes[pb][r, sl]
                return 0

            lax.fori_loop(0, CHUNK, rows, 0)
            write_copy(pc, bb, tb).start()

            ntb = (j + 2) % 4
            npc = 2 * i + (j + 2) // batch
            nbb = (j + 2) % batch

            @pl.when(g + 2 < steps)
            def _():
                @pl.when(g >= 2)
                def _():
                    opc = 2 * i + (j - 2) // batch
                    obb = (j - 2) % batch
                    write_copy(opc, obb, ntb).wait()
                gather_copy(npc, nbb, ntb).start()

            if bb == 2:
                @pl.when(pc + 1 < n_pc)
                def _():
                    pe_copy(pc + 1, 1 - pb).start()
        return 0

    lax.fori_loop(0, steps // (2 * batch), outer, 0)

    for j in range(4):
        pc = n_pc - 1
        bb = j % batch
        write_copy(pc, bb, j).wait()


@jax.jit
def kernel(token_ids, W_tok, pe):
    batch, seq_len = token_ids.shape
    n_rows = batch * seq_len
    ids = token_ids.reshape(-1).astype(jnp.int32)
    rows_per_worker = n_rows // NUM_WORKERS

    mesh = plsc.VectorSubcoreMesh(core_axis_name="c", subcore_axis_name="s")
    body = functools.partial(_emb_body, batch, seq_len)
    out = pl.kernel(
        body,
        mesh=mesh,
        out_type=jax.ShapeDtypeStruct((n_rows, D_MODEL), jnp.float32),
        scratch_types=(
            [pltpu.VMEM((rows_per_worker,), jnp.int32)]
            + [pltpu.VMEM((CHUNK, D_MODEL), jnp.float32) for _ in range(6)]
            + [pltpu.SemaphoreType.DMA for _ in range(10)]
        ),
    )(ids, W_tok, pe)
    return out.reshape(batch, seq_len, D_MODEL)


# speedup vs baseline: 3.2758x; 1.0089x over previous
import functools
import math

import jax
import jax.numpy as jnp
from jax import lax
from jax.experimental import pallas as pl
from jax.experimental.pallas import tpu as pltpu
from jax.experimental.pallas import tpu_sc as plsc

D_MODEL = 1024
LANES = 16
NUM_WORKERS = 32
CHUNK = 16
SCALE = math.sqrt(D_MODEL)


def _emb_body(batch, seq_len, ids_hbm, table_hbm, pe_hbm, out_hbm,
              idx_v, tok0, tok1, pe0, pe1, res0, res1,
              st0, st1, sp0, sp1, sw0, sw1):
    toks = (tok0, tok1)
    pes = (pe0, pe1)
    ress = (res0, res1)
    sts = (st0, st1)
    sps = (sp0, sp1)
    sws = (sw0, sw1)

    pos_per_worker = seq_len // NUM_WORKERS
    n_pc = pos_per_worker // CHUNK
    steps = n_pc * batch

    wid = lax.axis_index("s") * 2 + lax.axis_index("c")
    wpos = wid * pos_per_worker

    def gather_copy(pc, bb, tb):
        return pltpu.make_async_copy(
            table_hbm.at[idx_v.at[pl.ds(bb * pos_per_worker + pc * CHUNK,
                                        CHUNK)]],
            toks[tb], sts[tb])

    def pe_copy(pc, pb):
        return pltpu.make_async_copy(
            pe_hbm.at[pl.ds(wpos + pc * CHUNK, CHUNK)], pes[pb], sps[pb])

    def write_copy(pc, bb, ob):
        return pltpu.make_async_copy(
            ress[ob], out_hbm.at[pl.ds(bb * seq_len + wpos + pc * CHUNK,
                                       CHUNK)], sws[ob])

    def idx_copy(b, sem):
        return pltpu.make_async_copy(
            ids_hbm.at[pl.ds(b * seq_len + wpos, pos_per_worker)],
            idx_v.at[pl.ds(b * pos_per_worker, pos_per_worker)], sem)

    pe_copy(0, 0).start()
    idx_sems = (st0, st1, sw0, sw1)
    for b in range(batch):
        idx_copy(b, idx_sems[b]).start()
    for b in range(batch):
        idx_copy(b, idx_sems[b]).wait()
    gather_copy(0, 0, 0).start()
    gather_copy(0, 1, 1).start()

    def outer(i, _):
        for j in range(2 * batch):
            pc = 2 * i + j // batch
            bb = j % batch
            g = 2 * batch * i + j
            tb = j % 2
            ob = j % 2
            pb = (j // batch) % 2

            if bb == 0:
                pe_copy(pc, pb).wait()
            gather_copy(pc, bb, tb).wait()

            @pl.when(g >= 2)
            def _():
                opc = 2 * i + (j - 2) // batch
                obb = (j - 2) % batch
                write_copy(opc, obb, ob).wait()

            def rows(r, _):
                for c in range(D_MODEL // LANES):
                    sl = pl.ds(c * LANES, LANES)
                    ress[ob][r, sl] = (toks[tb][r, sl] * SCALE
                                       + pes[pb][r, sl])
                return 0

            lax.fori_loop(0, CHUNK, rows, 0)
            write_copy(pc, bb, ob).start()

            npc = 2 * i + (j + 2) // batch
            nbb = (j + 2) % batch

            @pl.when(g + 2 < steps)
            def _():
                gather_copy(npc, nbb, tb).start()

            if bb == 2:
                @pl.when(pc + 1 < n_pc)
                def _():
                    pe_copy(pc + 1, 1 - pb).start()
        return 0

    lax.fori_loop(0, steps // (2 * batch), outer, 0)

    write_copy(n_pc - 1, 2, 0).wait()
    write_copy(n_pc - 1, 3, 1).wait()


@jax.jit
def kernel(token_ids, W_tok, pe):
    batch, seq_len = token_ids.shape
    n_rows = batch * seq_len
    ids = token_ids.reshape(-1).astype(jnp.int32)
    rows_per_worker = n_rows // NUM_WORKERS

    mesh = plsc.VectorSubcoreMesh(core_axis_name="c", subcore_axis_name="s")
    body = functools.partial(_emb_body, batch, seq_len)
    out = pl.kernel(
        body,
        mesh=mesh,
        out_type=jax.ShapeDtypeStruct((n_rows, D_MODEL), jnp.float32),
        scratch_types=(
            [pltpu.VMEM((rows_per_worker,), jnp.int32)]
            + [pltpu.VMEM((CHUNK, D_MODEL), jnp.float32) for _ in range(6)]
            + [pltpu.SemaphoreType.DMA for _ in range(6)]
        ),
    )(ids, W_tok, pe)
    return out.reshape(batch, seq_len, D_MODEL)
